# baseline (speedup 1.0000x reference)
import functools

import jax
import jax.numpy as jnp
from jax.experimental import pallas as pl
from jax.experimental.pallas import tpu as pltpu

_D = 768
_MAX_REL = 125
_VOCAB = 2 * _MAX_REL + 1
_VPAD = 256
_BLK = 1024


def _body(x_ref, table_ref, out_ref, pooled_ref, *, S):
    s = pl.program_id(0)
    b = pl.program_id(1)
    blk = out_ref.shape[1]

    @pl.when(b == 0)
    def _compute_pooled():
        jj = s * blk + jax.lax.broadcasted_iota(jnp.int32, (blk, _VPAD), 0)
        kk = jax.lax.broadcasted_iota(jnp.int32, (blk, _VPAD), 1)
        interior = ((kk >= 1) & (kk <= _VOCAB - 2)
                    & (jj >= kk - _MAX_REL) & (jj <= kk - _MAX_REL + S - 1))
        cnt = jnp.where(kk == 0, jnp.maximum(0, (S - _MAX_REL) - jj), 0)
        cnt = cnt + jnp.where(kk == _VOCAB - 1,
                              jnp.maximum(0, jj - (_MAX_REL - 1)), 0)
        cnt = cnt + interior.astype(jnp.int32)
        c = cnt.astype(jnp.float32)
        pooled_ref[...] = jax.lax.dot_general(
            c, table_ref[...],
            dimension_numbers=(((1,), (0,)), ((), ())),
            preferred_element_type=jnp.float32,
        ) * (1.0 / S)

    out_ref[...] = x_ref[...] + pooled_ref[...][None, :, :]


def kernel(x, table):
    B, S, d = x.shape
    table_pad = jnp.zeros((_VPAD, d), dtype=table.dtype).at[:_VOCAB].set(table)
    grid = (S // _BLK, B)
    body = functools.partial(_body, S=S)
    return pl.pallas_call(
        body,
        grid=grid,
        in_specs=[
            pl.BlockSpec((1, _BLK, d), lambda s, b: (b, s, 0)),
            pl.BlockSpec((_VPAD, d), lambda s, b: (0, 0)),
        ],
        out_specs=pl.BlockSpec((1, _BLK, d), lambda s, b: (b, s, 0)),
        out_shape=jax.ShapeDtypeStruct((B, S, d), x.dtype),
        scratch_shapes=[pltpu.VMEM((_BLK, d), jnp.float32)],
    )(x, table_pad)


# speedup vs baseline: 1.4500x; 1.4500x over previous
import functools

import jax
import jax.numpy as jnp
from jax.experimental import pallas as pl

_D = 768
_MAX_REL = 125
_VOCAB = 2 * _MAX_REL + 1
_BLK = 1024


def _body(x_ref, table_ref, out_ref, *, S):
    s = pl.program_id(0)
    blk = out_ref.shape[1]
    kdim = table_ref.shape[0]
    jj = s * blk + jax.lax.broadcasted_iota(jnp.int32, (blk, kdim), 0)
    kk = jax.lax.broadcasted_iota(jnp.int32, (blk, kdim), 1)
    interior = ((kk >= 1) & (kk <= _VOCAB - 2)
                & (jj >= kk - _MAX_REL) & (jj <= kk - _MAX_REL + S - 1))
    cnt = jnp.where(kk == 0, jnp.maximum(0, (S - _MAX_REL) - jj), 0)
    cnt = cnt + jnp.where(kk == _VOCAB - 1, jnp.maximum(0, jj - (_MAX_REL - 1)), 0)
    cnt = cnt + interior.astype(jnp.int32)
    c = cnt.astype(jnp.float32)
    pooled = jax.lax.dot_general(
        c, table_ref[...],
        dimension_numbers=(((1,), (0,)), ((), ())),
        preferred_element_type=jnp.float32,
    ) * (1.0 / S)
    out_ref[...] = x_ref[...] + pooled[None, :, :]


def kernel(x, table):
    B, S, d = x.shape
    V = table.shape[0]
    grid = (S // _BLK,)
    body = functools.partial(_body, S=S)
    return pl.pallas_call(
        body,
        grid=grid,
        in_specs=[
            pl.BlockSpec((B, _BLK, d), lambda s: (0, s, 0)),
            pl.BlockSpec((V, d), lambda s: (0, 0)),
        ],
        out_specs=pl.BlockSpec((B, _BLK, d), lambda s: (0, s, 0)),
        out_shape=jax.ShapeDtypeStruct((B, S, d), x.dtype),
    )(x, table)
